# Initial kernel scaffold; baseline (speedup 1.0000x reference)
import jax
import jax.numpy as jnp
from jax.experimental import pallas as pl


def kernel(x, h0, numerators, denumerator):
    raise NotImplementedError("write your pallas kernel here")


# speedup vs baseline: 3.1136x; 3.1136x over previous
import jax
import jax.numpy as jnp
from jax.experimental import pallas as pl
from jax.experimental.pallas import tpu as pltpu

_L = 512
_D = 16
_NFH = 2 * _L
_NFW = _L + 1
_TH = 8


def _mix_kernel(xr_ref, xi_ref, hr_ref, hi_ref, h0_ref, yr_ref, yi_ref):
    xr = xr_ref[:]
    xi = xi_ref[:]
    h0 = h0_ref[:]
    accr = jnp.zeros(yr_ref.shape, dtype=jnp.float32)
    acci = jnp.zeros(yr_ref.shape, dtype=jnp.float32)
    for i in range(_D):
        xri = xr[:, i][:, None]
        xii = xi[:, i][:, None]
        hri = (hr_ref[i] + h0[i][:, None, None])[None]
        hii = hi_ref[i][None]
        accr = accr + xri * hri - xii * hii
        acci = acci + xri * hii + xii * hri
    yr_ref[:] = accr
    yi_ref[:] = acci


def _mix(Xr, Xi, Hr, Hi, h0):
    grid = (_NFH // _TH,)
    xspec = pl.BlockSpec((4, _D, _TH, _NFW), lambda g: (0, 0, g, 0))
    hspec = pl.BlockSpec((_D, _D, _TH, _NFW), lambda g: (0, 0, g, 0))
    return pl.pallas_call(
        _mix_kernel,
        grid=grid,
        in_specs=[xspec, xspec, hspec, hspec,
                  pl.BlockSpec((_D, _D), lambda g: (0, 0))],
        out_specs=[xspec, xspec],
        out_shape=[jax.ShapeDtypeStruct((4, _D, _NFH, _NFW), jnp.float32),
                   jax.ShapeDtypeStruct((4, _D, _NFH, _NFW), jnp.float32)],
        compiler_params=pltpu.CompilerParams(
            dimension_semantics=("parallel",),
        ),
    )(Xr, Xi, Hr, Hi, h0)


def kernel(x, h0, numerators, denumerator):
    l1, l2 = x.shape[-3], x.shape[-2]
    a = jnp.pad(denumerator, ((0, 0), (0, 0), (1, 0), (1, 0)), constant_values=1.0)
    b0 = jnp.pad(numerators[0], ((0, 0), (0, 0), (1, 0), (1, 0)), constant_values=0.0)
    Ht = jnp.fft.rfft2(b0, s=(l1, l2)) / jnp.fft.rfft2(a, s=(l1, l2))
    h_t = jnp.fft.irfft2(Ht, s=(l1, l2))
    H = jnp.fft.rfft2(h_t, s=(2 * l1, 2 * l2))
    xt = jnp.transpose(x, (0, 3, 1, 2))
    X = jnp.fft.rfft2(xt, s=(2 * l1, 2 * l2))
    Yr, Yi = _mix(X.real, X.imag, H.real, H.imag, h0[:, :, 0, 0])
    Y = jax.lax.complex(Yr, Yi)
    y_full = jnp.fft.irfft2(Y, s=(2 * l1, 2 * l2))
    y = y_full[..., :l1, :l2]
    return jnp.transpose(y, (0, 2, 3, 1))


# speedup vs baseline: 4.5817x; 1.4715x over previous
import jax
import jax.numpy as jnp
from jax.experimental import pallas as pl
from jax.experimental.pallas import tpu as pltpu

_L = 512
_D = 16
_NFH = 2 * _L
_NFW = _L + 1
_TH = 8
_PREC = jax.lax.Precision.HIGHEST


def _trig(K, N, M):
    k = jnp.arange(K, dtype=jnp.int32)[:, None]
    n = jnp.arange(N, dtype=jnp.int32)[None, :]
    ph = (k * n) % M
    th = ph.astype(jnp.float32) * jnp.float32(2.0 * jnp.pi / M)
    return jnp.cos(th), -jnp.sin(th)


def _cmm(ar, ai, br, bi):
    rr = jnp.dot(ar, br, precision=_PREC) - jnp.dot(ai, bi, precision=_PREC)
    ri = jnp.dot(ar, bi, precision=_PREC) + jnp.dot(ai, br, precision=_PREC)
    return rr, ri


def _hside_kernel(htr, hti, m1r, m1i, m2r, m2i, hr_out, hi_out):
    ar, ai = htr[0], hti[0]
    tr, ti = _cmm(m1r[:], m1i[:], ar, ai)
    rr, ri = _cmm(tr, ti, m2r[:], m2i[:])
    hr_out[0] = rr
    hi_out[0] = ri


def _hside(htr, hti, m1r, m1i, m2r, m2i):
    n = htr.shape[0]
    cspec = lambda shape: pl.BlockSpec(shape, lambda g: (0, 0))
    pspec = pl.BlockSpec((1, _L, _L), lambda g: (g, 0, 0))
    ospec = pl.BlockSpec((1, _NFH, _NFW), lambda g: (g, 0, 0))
    return pl.pallas_call(
        _hside_kernel,
        grid=(n,),
        in_specs=[pspec, pspec,
                  cspec((_NFH, _L)), cspec((_NFH, _L)),
                  cspec((_L, _NFW)), cspec((_L, _NFW))],
        out_specs=[ospec, ospec],
        out_shape=[jax.ShapeDtypeStruct((n, _NFH, _NFW), jnp.float32)] * 2,
        compiler_params=pltpu.CompilerParams(
            dimension_semantics=("parallel",)),
    )(htr, hti, m1r, m1i, m2r, m2i)


def _xside_kernel(x_ref, rr_ref, ri_ref, cr_ref, ci_ref, xr_out, xi_out):
    xv = x_ref[0]
    tr = jnp.dot(xv, cr_ref[:], precision=_PREC)
    ti = jnp.dot(xv, ci_ref[:], precision=_PREC)
    rr, ri = _cmm(rr_ref[:], ri_ref[:], tr, ti)
    xr_out[0] = rr
    xi_out[0] = ri


def _xside(xt, rr, ri, cr, ci):
    n = xt.shape[0]
    cspec = lambda shape: pl.BlockSpec(shape, lambda g: (0, 0))
    return pl.pallas_call(
        _xside_kernel,
        grid=(n,),
        in_specs=[pl.BlockSpec((1, _L, _L), lambda g: (g, 0, 0)),
                  cspec((_NFH, _L)), cspec((_NFH, _L)),
                  cspec((_L, _NFW)), cspec((_L, _NFW))],
        out_specs=[pl.BlockSpec((1, _NFH, _NFW), lambda g: (g, 0, 0))] * 2,
        out_shape=[jax.ShapeDtypeStruct((n, _NFH, _NFW), jnp.float32)] * 2,
        compiler_params=pltpu.CompilerParams(
            dimension_semantics=("parallel",)),
    )(xt, rr, ri, cr, ci)


def _mix_kernel(xr_ref, xi_ref, hr_ref, hi_ref, h0_ref, yr_ref, yi_ref):
    xr = xr_ref[:]
    xi = xi_ref[:]
    h0 = h0_ref[:]
    accr = jnp.zeros(yr_ref.shape, dtype=jnp.float32)
    acci = jnp.zeros(yr_ref.shape, dtype=jnp.float32)
    for i in range(_D):
        xri = xr[:, i][:, None]
        xii = xi[:, i][:, None]
        hri = (hr_ref[i] + h0[i][:, None, None])[None]
        hii = hi_ref[i][None]
        accr = accr + xri * hri - xii * hii
        acci = acci + xri * hii + xii * hri
    yr_ref[:] = accr
    yi_ref[:] = acci


def _mix(Xr, Xi, Hr, Hi, h0):
    xspec = pl.BlockSpec((4, _D, _TH, _NFW), lambda g: (0, 0, g, 0))
    hspec = pl.BlockSpec((_D, _D, _TH, _NFW), lambda g: (0, 0, g, 0))
    return pl.pallas_call(
        _mix_kernel,
        grid=(_NFH // _TH,),
        in_specs=[xspec, xspec, hspec, hspec,
                  pl.BlockSpec((_D, _D), lambda g: (0, 0))],
        out_specs=[xspec, xspec],
        out_shape=[jax.ShapeDtypeStruct((4, _D, _NFH, _NFW), jnp.float32)] * 2,
        compiler_params=pltpu.CompilerParams(
            dimension_semantics=("parallel",)),
    )(Xr, Xi, Hr, Hi, h0)


def _inv_kernel(yr_ref, yi_ref, vr_ref, vi_ref, wr_ref, wi_ref, y_out):
    yr, yi = yr_ref[0], yi_ref[0]
    zr, zi = _cmm(vr_ref[:], vi_ref[:], yr, yi)
    y_out[0] = (jnp.dot(zr, wr_ref[:], precision=_PREC)
                - jnp.dot(zi, wi_ref[:], precision=_PREC))


def _inv(Yr, Yi, vr, vi, wr, wi):
    n = Yr.shape[0]
    cspec = lambda shape: pl.BlockSpec(shape, lambda g: (0, 0))
    return pl.pallas_call(
        _inv_kernel,
        grid=(n,),
        in_specs=[pl.BlockSpec((1, _NFH, _NFW), lambda g: (g, 0, 0))] * 2
                 + [cspec((_L, _NFH)), cspec((_L, _NFH)),
                    cspec((_NFW, _L)), cspec((_NFW, _L))],
        out_specs=pl.BlockSpec((1, _L, _L), lambda g: (g, 0, 0)),
        out_shape=jax.ShapeDtypeStruct((n, _L, _L), jnp.float32),
        compiler_params=pltpu.CompilerParams(
            dimension_semantics=("parallel",)),
    )(Yr, Yi, vr, vi, wr, wi)


def _constants():
    e_r, e_i = _trig(_NFH, _L, _NFH)
    f_r, f_i = _trig(_L, _L, _L)
    inv_r, inv_i = f_r / _L, -f_i / _L
    m1r, m1i = _cmm(e_r, e_i, inv_r, inv_i)
    c_r, c_i = _trig(_L, _NFW, _NFH)
    m2r, m2i = _cmm(inv_r, inv_i, c_r, c_i)
    v_r, v_i = _trig(_L, _NFH, _NFH)
    v_i = -v_i
    w_r, w_i = _trig(_NFW, _L, _NFH)
    w_i = -w_i
    cm = jnp.full((_NFW, 1), 2.0, jnp.float32)
    cm = cm.at[0, 0].set(1.0).at[_NFW - 1, 0].set(1.0) / float(_NFH * _NFH)
    w_r = w_r * cm
    w_i = w_i * cm
    return e_r, e_i, c_r, c_i, m1r, m1i, m2r, m2i, v_r, v_i, w_r, w_i


def kernel(x, h0, numerators, denumerator):
    l1, l2 = x.shape[-3], x.shape[-2]
    a = jnp.pad(denumerator, ((0, 0), (0, 0), (1, 0), (1, 0)), constant_values=1.0)
    b0 = jnp.pad(numerators[0], ((0, 0), (0, 0), (1, 0), (1, 0)), constant_values=0.0)
    Ht = jnp.fft.rfft2(b0, s=(l1, l2)) / jnp.fft.rfft2(a, s=(l1, l2))
    Ht = Ht.reshape(_D * _D, l1, l2 // 2 + 1)
    mirror = jnp.roll(jnp.flip(Ht, axis=1), 1, axis=1)
    ext = jnp.flip(jnp.conj(mirror[:, :, 1:_L // 2]), axis=2)
    Htf = jnp.concatenate([Ht, ext], axis=2)

    er, ei, cr, ci, m1r, m1i, m2r, m2i, vr, vi, wr, wi = _constants()
    Hr, Hi = _hside(jnp.real(Htf), jnp.imag(Htf), m1r, m1i, m2r, m2i)

    xt = jnp.transpose(x, (0, 3, 1, 2)).reshape(4 * _D, l1, l2)
    Xr, Xi = _xside(xt, er, ei, cr, ci)

    Yr, Yi = _mix(Xr.reshape(4, _D, _NFH, _NFW), Xi.reshape(4, _D, _NFH, _NFW),
                  Hr.reshape(_D, _D, _NFH, _NFW), Hi.reshape(_D, _D, _NFH, _NFW),
                  h0[:, :, 0, 0])

    y = _inv(Yr.reshape(4 * _D, _NFH, _NFW), Yi.reshape(4 * _D, _NFH, _NFW),
             vr, vi, wr, wi)
    return jnp.transpose(y.reshape(4, _D, l1, l2), (0, 2, 3, 1))


# speedup vs baseline: 6.3354x; 1.3827x over previous
import jax
import jax.numpy as jnp
from jax.experimental import pallas as pl
from jax.experimental.pallas import tpu as pltpu

_L = 512
_D = 16
_NFH = 2 * _L
_NFW = _L + 1
_TH = 8
_PREC = jax.lax.Precision.DEFAULT


def _trig(K, N, M):
    k = jnp.arange(K, dtype=jnp.int32)[:, None]
    n = jnp.arange(N, dtype=jnp.int32)[None, :]
    ph = (k * n) % M
    th = ph.astype(jnp.float32) * jnp.float32(2.0 * jnp.pi / M)
    return jnp.cos(th), -jnp.sin(th)


def _cmm(ar, ai, br, bi):
    rr = jnp.dot(ar, br, precision=_PREC) - jnp.dot(ai, bi, precision=_PREC)
    ri = jnp.dot(ar, bi, precision=_PREC) + jnp.dot(ai, br, precision=_PREC)
    return rr, ri


def _hside_kernel(htr, hti, m1r, m1i, m2r, m2i, hr_out, hi_out):
    ar, ai = htr[0], hti[0]
    tr, ti = _cmm(m1r[:], m1i[:], ar, ai)
    rr, ri = _cmm(tr, ti, m2r[:], m2i[:])
    hr_out[0] = rr
    hi_out[0] = ri


def _hside(htr, hti, m1r, m1i, m2r, m2i):
    n = htr.shape[0]
    cspec = lambda shape: pl.BlockSpec(shape, lambda g: (0, 0))
    pspec = pl.BlockSpec((1, _L, _L), lambda g: (g, 0, 0))
    ospec = pl.BlockSpec((1, _NFH, _NFW), lambda g: (g, 0, 0))
    return pl.pallas_call(
        _hside_kernel,
        grid=(n,),
        in_specs=[pspec, pspec,
                  cspec((_NFH, _L)), cspec((_NFH, _L)),
                  cspec((_L, _NFW)), cspec((_L, _NFW))],
        out_specs=[ospec, ospec],
        out_shape=[jax.ShapeDtypeStruct((n, _NFH, _NFW), jnp.float32)] * 2,
        compiler_params=pltpu.CompilerParams(
            dimension_semantics=("parallel",)),
    )(htr, hti, m1r, m1i, m2r, m2i)


def _xside_kernel(x_ref, rr_ref, ri_ref, cr_ref, ci_ref, xr_out, xi_out):
    xv = x_ref[0]
    tr = jnp.dot(xv, cr_ref[:], precision=_PREC)
    ti = jnp.dot(xv, ci_ref[:], precision=_PREC)
    rr, ri = _cmm(rr_ref[:], ri_ref[:], tr, ti)
    xr_out[0] = rr
    xi_out[0] = ri


def _xside(xt, rr, ri, cr, ci):
    n = xt.shape[0]
    cspec = lambda shape: pl.BlockSpec(shape, lambda g: (0, 0))
    return pl.pallas_call(
        _xside_kernel,
        grid=(n,),
        in_specs=[pl.BlockSpec((1, _L, _L), lambda g: (g, 0, 0)),
                  cspec((_NFH, _L)), cspec((_NFH, _L)),
                  cspec((_L, _NFW)), cspec((_L, _NFW))],
        out_specs=[pl.BlockSpec((1, _NFH, _NFW), lambda g: (g, 0, 0))] * 2,
        out_shape=[jax.ShapeDtypeStruct((n, _NFH, _NFW), jnp.float32)] * 2,
        compiler_params=pltpu.CompilerParams(
            dimension_semantics=("parallel",)),
    )(xt, rr, ri, cr, ci)


def _mix_kernel(xr_ref, xi_ref, hr_ref, hi_ref, h0_ref, yr_ref, yi_ref):
    xr = xr_ref[:]
    xi = xi_ref[:]
    h0 = h0_ref[:]
    accr = jnp.zeros(yr_ref.shape, dtype=jnp.float32)
    acci = jnp.zeros(yr_ref.shape, dtype=jnp.float32)
    for i in range(_D):
        xri = xr[:, i][:, None]
        xii = xi[:, i][:, None]
        hri = (hr_ref[i] + h0[i][:, None, None])[None]
        hii = hi_ref[i][None]
        accr = accr + xri * hri - xii * hii
        acci = acci + xri * hii + xii * hri
    yr_ref[:] = accr
    yi_ref[:] = acci


def _mix(Xr, Xi, Hr, Hi, h0):
    xspec = pl.BlockSpec((4, _D, _TH, _NFW), lambda g: (0, 0, g, 0))
    hspec = pl.BlockSpec((_D, _D, _TH, _NFW), lambda g: (0, 0, g, 0))
    return pl.pallas_call(
        _mix_kernel,
        grid=(_NFH // _TH,),
        in_specs=[xspec, xspec, hspec, hspec,
                  pl.BlockSpec((_D, _D), lambda g: (0, 0))],
        out_specs=[xspec, xspec],
        out_shape=[jax.ShapeDtypeStruct((4, _D, _NFH, _NFW), jnp.float32)] * 2,
        compiler_params=pltpu.CompilerParams(
            dimension_semantics=("parallel",)),
    )(Xr, Xi, Hr, Hi, h0)


def _inv_kernel(yr_ref, yi_ref, vr_ref, vi_ref, wr_ref, wi_ref, y_out):
    yr, yi = yr_ref[0], yi_ref[0]
    zr, zi = _cmm(vr_ref[:], vi_ref[:], yr, yi)
    y_out[0] = (jnp.dot(zr, wr_ref[:], precision=_PREC)
                - jnp.dot(zi, wi_ref[:], precision=_PREC))


def _inv(Yr, Yi, vr, vi, wr, wi):
    n = Yr.shape[0]
    cspec = lambda shape: pl.BlockSpec(shape, lambda g: (0, 0))
    return pl.pallas_call(
        _inv_kernel,
        grid=(n,),
        in_specs=[pl.BlockSpec((1, _NFH, _NFW), lambda g: (g, 0, 0))] * 2
                 + [cspec((_L, _NFH)), cspec((_L, _NFH)),
                    cspec((_NFW, _L)), cspec((_NFW, _L))],
        out_specs=pl.BlockSpec((1, _L, _L), lambda g: (g, 0, 0)),
        out_shape=jax.ShapeDtypeStruct((n, _L, _L), jnp.float32),
        compiler_params=pltpu.CompilerParams(
            dimension_semantics=("parallel",)),
    )(Yr, Yi, vr, vi, wr, wi)


def _constants():
    e_r, e_i = _trig(_NFH, _L, _NFH)
    f_r, f_i = _trig(_L, _L, _L)
    inv_r, inv_i = f_r / _L, -f_i / _L
    m1r, m1i = _cmm(e_r, e_i, inv_r, inv_i)
    c_r, c_i = _trig(_L, _NFW, _NFH)
    m2r, m2i = _cmm(inv_r, inv_i, c_r, c_i)
    v_r, v_i = _trig(_L, _NFH, _NFH)
    v_i = -v_i
    w_r, w_i = _trig(_NFW, _L, _NFH)
    w_i = -w_i
    cm = jnp.full((_NFW, 1), 2.0, jnp.float32)
    cm = cm.at[0, 0].set(1.0).at[_NFW - 1, 0].set(1.0) / float(_NFH * _NFH)
    w_r = w_r * cm
    w_i = w_i * cm
    return e_r, e_i, c_r, c_i, m1r, m1i, m2r, m2i, v_r, v_i, w_r, w_i


def kernel(x, h0, numerators, denumerator):
    l1, l2 = x.shape[-3], x.shape[-2]
    a = jnp.pad(denumerator, ((0, 0), (0, 0), (1, 0), (1, 0)), constant_values=1.0)
    b0 = jnp.pad(numerators[0], ((0, 0), (0, 0), (1, 0), (1, 0)), constant_values=0.0)
    Ht = jnp.fft.rfft2(b0, s=(l1, l2)) / jnp.fft.rfft2(a, s=(l1, l2))
    Ht = Ht.reshape(_D * _D, l1, l2 // 2 + 1)
    mirror = jnp.roll(jnp.flip(Ht, axis=1), 1, axis=1)
    ext = jnp.flip(jnp.conj(mirror[:, :, 1:_L // 2]), axis=2)
    Htf = jnp.concatenate([Ht, ext], axis=2)

    er, ei, cr, ci, m1r, m1i, m2r, m2i, vr, vi, wr, wi = _constants()
    Hr, Hi = _hside(jnp.real(Htf), jnp.imag(Htf), m1r, m1i, m2r, m2i)

    xt = jnp.transpose(x, (0, 3, 1, 2)).reshape(4 * _D, l1, l2)
    Xr, Xi = _xside(xt, er, ei, cr, ci)

    Yr, Yi = _mix(Xr.reshape(4, _D, _NFH, _NFW), Xi.reshape(4, _D, _NFH, _NFW),
                  Hr.reshape(_D, _D, _NFH, _NFW), Hi.reshape(_D, _D, _NFH, _NFW),
                  h0[:, :, 0, 0])

    y = _inv(Yr.reshape(4 * _D, _NFH, _NFW), Yi.reshape(4 * _D, _NFH, _NFW),
             vr, vi, wr, wi)
    return jnp.transpose(y.reshape(4, _D, l1, l2), (0, 2, 3, 1))


# speedup vs baseline: 13.7985x; 2.1780x over previous
import jax
import jax.numpy as jnp
from jax.experimental import pallas as pl
from jax.experimental.pallas import tpu as pltpu

_L = 512
_D = 16
_NFH = 2 * _L
_NFW = _L + 1
_TH = 8
_PREC = jax.lax.Precision.DEFAULT


def _trig(K, N, M):
    k = jnp.arange(K, dtype=jnp.int32)[:, None]
    n = jnp.arange(N, dtype=jnp.int32)[None, :]
    ph = (k * n) % M
    th = ph.astype(jnp.float32) * jnp.float32(2.0 * jnp.pi / M)
    return jnp.cos(th), -jnp.sin(th)


def _cmm(ar, ai, br, bi):
    rr = jnp.dot(ar, br, precision=_PREC) - jnp.dot(ai, bi, precision=_PREC)
    ri = jnp.dot(ar, bi, precision=_PREC) + jnp.dot(ai, br, precision=_PREC)
    return rr, ri


def _tf_kernel(a_ref, b_ref, w1r, w1i, w2r, w2i, htr_out, hti_out):
    hp = jax.lax.Precision.HIGHEST
    av, bv = a_ref[0], b_ref[0]
    tar = jnp.dot(w1r[:], av, precision=hp)
    tai = jnp.dot(w1i[:], av, precision=hp)
    tbr = jnp.dot(w1r[:], bv, precision=hp)
    tbi = jnp.dot(w1i[:], bv, precision=hp)
    afr = jnp.dot(tar, w2r[:], precision=hp) - jnp.dot(tai, w2i[:], precision=hp)
    afi = jnp.dot(tar, w2i[:], precision=hp) + jnp.dot(tai, w2r[:], precision=hp)
    bfr = jnp.dot(tbr, w2r[:], precision=hp) - jnp.dot(tbi, w2i[:], precision=hp)
    bfi = jnp.dot(tbr, w2i[:], precision=hp) + jnp.dot(tbi, w2r[:], precision=hp)
    d = afr * afr + afi * afi
    htr_out[0] = (bfr * afr + bfi * afi) / d
    hti_out[0] = (bfi * afr - bfr * afi) / d


def _tf(apad, bpad, w1r, w1i, w2r, w2i):
    n = apad.shape[0]
    cspec = lambda shape: pl.BlockSpec(shape, lambda g: (0, 0))
    hw = _L // 2 + 1
    return pl.pallas_call(
        _tf_kernel,
        grid=(n,),
        in_specs=[pl.BlockSpec((1, 16, 16), lambda g: (g, 0, 0))] * 2
                 + [cspec((_L, 16)), cspec((_L, 16)),
                    cspec((16, hw)), cspec((16, hw))],
        out_specs=[pl.BlockSpec((1, _L, hw), lambda g: (g, 0, 0))] * 2,
        out_shape=[jax.ShapeDtypeStruct((n, _L, hw), jnp.float32)] * 2,
        compiler_params=pltpu.CompilerParams(
            dimension_semantics=("parallel",)),
    )(apad, bpad, w1r, w1i, w2r, w2i)


def _hside_kernel(htr, hti, m1r, m1i, m2r, m2i, hr_out, hi_out):
    ar, ai = htr[0], hti[0]
    tr, ti = _cmm(m1r[:], m1i[:], ar, ai)
    rr, ri = _cmm(tr, ti, m2r[:], m2i[:])
    hr_out[0] = rr
    hi_out[0] = ri


def _hside(htr, hti, m1r, m1i, m2r, m2i):
    n = htr.shape[0]
    cspec = lambda shape: pl.BlockSpec(shape, lambda g: (0, 0))
    pspec = pl.BlockSpec((1, _L, _L), lambda g: (g, 0, 0))
    ospec = pl.BlockSpec((1, _NFH, _NFW), lambda g: (g, 0, 0))
    return pl.pallas_call(
        _hside_kernel,
        grid=(n,),
        in_specs=[pspec, pspec,
                  cspec((_NFH, _L)), cspec((_NFH, _L)),
                  cspec((_L, _NFW)), cspec((_L, _NFW))],
        out_specs=[ospec, ospec],
        out_shape=[jax.ShapeDtypeStruct((n, _NFH, _NFW), jnp.float32)] * 2,
        compiler_params=pltpu.CompilerParams(
            dimension_semantics=("parallel",)),
    )(htr, hti, m1r, m1i, m2r, m2i)


def _xside_kernel(x_ref, rr_ref, ri_ref, cr_ref, ci_ref, xr_out, xi_out):
    xv = x_ref[0]
    tr = jnp.dot(xv, cr_ref[:], precision=_PREC)
    ti = jnp.dot(xv, ci_ref[:], precision=_PREC)
    rr, ri = _cmm(rr_ref[:], ri_ref[:], tr, ti)
    xr_out[0] = rr
    xi_out[0] = ri


def _xside(xt, rr, ri, cr, ci):
    n = xt.shape[0]
    cspec = lambda shape: pl.BlockSpec(shape, lambda g: (0, 0))
    return pl.pallas_call(
        _xside_kernel,
        grid=(n,),
        in_specs=[pl.BlockSpec((1, _L, _L), lambda g: (g, 0, 0)),
                  cspec((_NFH, _L)), cspec((_NFH, _L)),
                  cspec((_L, _NFW)), cspec((_L, _NFW))],
        out_specs=[pl.BlockSpec((1, _NFH, _NFW), lambda g: (g, 0, 0))] * 2,
        out_shape=[jax.ShapeDtypeStruct((n, _NFH, _NFW), jnp.float32)] * 2,
        compiler_params=pltpu.CompilerParams(
            dimension_semantics=("parallel",)),
    )(xt, rr, ri, cr, ci)


def _mix_kernel(xr_ref, xi_ref, hr_ref, hi_ref, h0_ref, yr_ref, yi_ref):
    xr = xr_ref[:]
    xi = xi_ref[:]
    h0 = h0_ref[:]
    accr = jnp.zeros(yr_ref.shape, dtype=jnp.float32)
    acci = jnp.zeros(yr_ref.shape, dtype=jnp.float32)
    for i in range(_D):
        xri = xr[:, i][:, None]
        xii = xi[:, i][:, None]
        hri = (hr_ref[i] + h0[i][:, None, None])[None]
        hii = hi_ref[i][None]
        accr = accr + xri * hri - xii * hii
        acci = acci + xri * hii + xii * hri
    yr_ref[:] = accr
    yi_ref[:] = acci


def _mix(Xr, Xi, Hr, Hi, h0):
    xspec = pl.BlockSpec((4, _D, _TH, _NFW), lambda g: (0, 0, g, 0))
    hspec = pl.BlockSpec((_D, _D, _TH, _NFW), lambda g: (0, 0, g, 0))
    return pl.pallas_call(
        _mix_kernel,
        grid=(_NFH // _TH,),
        in_specs=[xspec, xspec, hspec, hspec,
                  pl.BlockSpec((_D, _D), lambda g: (0, 0))],
        out_specs=[xspec, xspec],
        out_shape=[jax.ShapeDtypeStruct((4, _D, _NFH, _NFW), jnp.float32)] * 2,
        compiler_params=pltpu.CompilerParams(
            dimension_semantics=("parallel",)),
    )(Xr, Xi, Hr, Hi, h0)


def _inv_kernel(yr_ref, yi_ref, vr_ref, vi_ref, wr_ref, wi_ref, y_out):
    yr, yi = yr_ref[0], yi_ref[0]
    zr, zi = _cmm(vr_ref[:], vi_ref[:], yr, yi)
    y_out[0] = (jnp.dot(zr, wr_ref[:], precision=_PREC)
                - jnp.dot(zi, wi_ref[:], precision=_PREC))


def _inv(Yr, Yi, vr, vi, wr, wi):
    n = Yr.shape[0]
    cspec = lambda shape: pl.BlockSpec(shape, lambda g: (0, 0))
    return pl.pallas_call(
        _inv_kernel,
        grid=(n,),
        in_specs=[pl.BlockSpec((1, _NFH, _NFW), lambda g: (g, 0, 0))] * 2
                 + [cspec((_L, _NFH)), cspec((_L, _NFH)),
                    cspec((_NFW, _L)), cspec((_NFW, _L))],
        out_specs=pl.BlockSpec((1, _L, _L), lambda g: (g, 0, 0)),
        out_shape=jax.ShapeDtypeStruct((n, _L, _L), jnp.float32),
        compiler_params=pltpu.CompilerParams(
            dimension_semantics=("parallel",)),
    )(Yr, Yi, vr, vi, wr, wi)


def _constants():
    e_r, e_i = _trig(_NFH, _L, _NFH)
    f_r, f_i = _trig(_L, _L, _L)
    inv_r, inv_i = f_r / _L, -f_i / _L
    m1r, m1i = _cmm(e_r, e_i, inv_r, inv_i)
    c_r, c_i = _trig(_L, _NFW, _NFH)
    m2r, m2i = _cmm(inv_r, inv_i, c_r, c_i)
    v_r, v_i = _trig(_L, _NFH, _NFH)
    v_i = -v_i
    w_r, w_i = _trig(_NFW, _L, _NFH)
    w_i = -w_i
    cm = jnp.full((_NFW, 1), 2.0, jnp.float32)
    cm = cm.at[0, 0].set(1.0).at[_NFW - 1, 0].set(1.0) / float(_NFH * _NFH)
    w_r = w_r * cm
    w_i = w_i * cm
    w1_r, w1_i = _trig(_L, 16, _L)
    w2_r, w2_i = _trig(16, _L // 2 + 1, _L)
    return (e_r, e_i, c_r, c_i, m1r, m1i, m2r, m2i, v_r, v_i, w_r, w_i,
            w1_r, w1_i, w2_r, w2_i)


def kernel(x, h0, numerators, denumerator):
    l1, l2 = x.shape[-3], x.shape[-2]
    a = jnp.pad(denumerator, ((0, 0), (0, 0), (1, 0), (1, 0)), constant_values=1.0)
    b0 = jnp.pad(numerators[0], ((0, 0), (0, 0), (1, 0), (1, 0)), constant_values=0.0)
    apad = jnp.pad(a.reshape(_D * _D, 9, 9), ((0, 0), (0, 7), (0, 7)))
    bpad = jnp.pad(b0.reshape(_D * _D, 9, 9), ((0, 0), (0, 7), (0, 7)))

    (er, ei, cr, ci, m1r, m1i, m2r, m2i, vr, vi, wr, wi,
     w1r, w1i, w2r, w2i) = _constants()
    Htr, Hti = _tf(apad, bpad, w1r, w1i, w2r, w2i)
    def _ext(p, sign):
        mirror = jnp.roll(jnp.flip(p, axis=1), 1, axis=1)
        return jnp.concatenate(
            [p, sign * jnp.flip(mirror[:, :, 1:_L // 2], axis=2)], axis=2)
    Htfr = _ext(Htr, 1.0)
    Htfi = _ext(Hti, -1.0)
    Hr, Hi = _hside(Htfr, Htfi, m1r, m1i, m2r, m2i)

    xt = jnp.transpose(x, (0, 3, 1, 2)).reshape(4 * _D, l1, l2)
    Xr, Xi = _xside(xt, er, ei, cr, ci)

    Yr, Yi = _mix(Xr.reshape(4, _D, _NFH, _NFW), Xi.reshape(4, _D, _NFH, _NFW),
                  Hr.reshape(_D, _D, _NFH, _NFW), Hi.reshape(_D, _D, _NFH, _NFW),
                  h0[:, :, 0, 0])

    y = _inv(Yr.reshape(4 * _D, _NFH, _NFW), Yi.reshape(4 * _D, _NFH, _NFW),
             vr, vi, wr, wi)
    return jnp.transpose(y.reshape(4, _D, l1, l2), (0, 2, 3, 1))


# speedup vs baseline: 14.4556x; 1.0476x over previous
import jax
import jax.numpy as jnp
from jax.experimental import pallas as pl
from jax.experimental.pallas import tpu as pltpu

_L = 512
_D = 16
_NFH = 2 * _L
_NFW = _L + 1
_TH = 8
_PREC = jax.lax.Precision.DEFAULT


def _trig(K, N, M):
    k = jnp.arange(K, dtype=jnp.int32)[:, None]
    n = jnp.arange(N, dtype=jnp.int32)[None, :]
    ph = (k * n) % M
    th = ph.astype(jnp.float32) * jnp.float32(2.0 * jnp.pi / M)
    return jnp.cos(th), -jnp.sin(th)


def _cmm(ar, ai, br, bi):
    rr = jnp.dot(ar, br, precision=_PREC) - jnp.dot(ai, bi, precision=_PREC)
    ri = jnp.dot(ar, bi, precision=_PREC) + jnp.dot(ai, br, precision=_PREC)
    return rr, ri


def _tf_kernel(a_ref, b_ref, w1r, w1i, w2r, w2i, htr_out, hti_out):
    hp = jax.lax.Precision.HIGHEST
    av, bv = a_ref[0], b_ref[0]
    tar = jnp.dot(w1r[:], av, precision=hp)
    tai = jnp.dot(w1i[:], av, precision=hp)
    tbr = jnp.dot(w1r[:], bv, precision=hp)
    tbi = jnp.dot(w1i[:], bv, precision=hp)
    afr = jnp.dot(tar, w2r[:], precision=hp) - jnp.dot(tai, w2i[:], precision=hp)
    afi = jnp.dot(tar, w2i[:], precision=hp) + jnp.dot(tai, w2r[:], precision=hp)
    bfr = jnp.dot(tbr, w2r[:], precision=hp) - jnp.dot(tbi, w2i[:], precision=hp)
    bfi = jnp.dot(tbr, w2i[:], precision=hp) + jnp.dot(tbi, w2r[:], precision=hp)
    d = afr * afr + afi * afi
    htr_out[0] = (bfr * afr + bfi * afi) / d
    hti_out[0] = (bfi * afr - bfr * afi) / d


def _tf(apad, bpad, w1r, w1i, w2r, w2i):
    n = apad.shape[0]
    cspec = lambda shape: pl.BlockSpec(shape, lambda g: (0, 0))
    hw = _L // 2 + 1
    return pl.pallas_call(
        _tf_kernel,
        grid=(n,),
        in_specs=[pl.BlockSpec((1, 16, 16), lambda g: (g, 0, 0))] * 2
                 + [cspec((_L, 16)), cspec((_L, 16)),
                    cspec((16, hw)), cspec((16, hw))],
        out_specs=[pl.BlockSpec((1, _L, hw), lambda g: (g, 0, 0))] * 2,
        out_shape=[jax.ShapeDtypeStruct((n, _L, hw), jnp.float32)] * 2,
        compiler_params=pltpu.CompilerParams(
            dimension_semantics=("parallel",)),
    )(apad, bpad, w1r, w1i, w2r, w2i)


def _hside_kernel(htr, hti, c1r, c1i, m2r, m2i, hr_out, hi_out):
    ar, ai = htr[0], hti[0]
    er, ei = _cmm(ar, ai, m2r[:], m2i[:])
    orr, oi = _cmm(c1r[:], c1i[:], er, ei)
    hr_out[0, :_L] = er
    hi_out[0, :_L] = ei
    hr_out[0, _L:] = orr
    hi_out[0, _L:] = oi


def _hside(htr, hti, c1r, c1i, m2r, m2i):
    n = htr.shape[0]
    cspec = lambda shape: pl.BlockSpec(shape, lambda g: (0, 0))
    pspec = pl.BlockSpec((1, _L, _L), lambda g: (g, 0, 0))
    ospec = pl.BlockSpec((1, _NFH, _NFW), lambda g: (g, 0, 0))
    return pl.pallas_call(
        _hside_kernel,
        grid=(n,),
        in_specs=[pspec, pspec,
                  cspec((_L, _L)), cspec((_L, _L)),
                  cspec((_L, _NFW)), cspec((_L, _NFW))],
        out_specs=[ospec, ospec],
        out_shape=[jax.ShapeDtypeStruct((n, _NFH, _NFW), jnp.float32)] * 2,
        compiler_params=pltpu.CompilerParams(
            dimension_semantics=("parallel",)),
    )(htr, hti, c1r, c1i, m2r, m2i)


def _xside_kernel(x_ref, rr_ref, ri_ref, cr_ref, ci_ref, xr_out, xi_out):
    xv = x_ref[0]
    tr = jnp.dot(xv, cr_ref[:], precision=_PREC)
    ti = jnp.dot(xv, ci_ref[:], precision=_PREC)
    rr, ri = _cmm(rr_ref[:], ri_ref[:], tr, ti)
    xr_out[0] = rr
    xi_out[0] = ri


def _xside(xt, rr, ri, cr, ci):
    n = xt.shape[0]
    cspec = lambda shape: pl.BlockSpec(shape, lambda g: (0, 0))
    return pl.pallas_call(
        _xside_kernel,
        grid=(n,),
        in_specs=[pl.BlockSpec((1, _L, _L), lambda g: (g, 0, 0)),
                  cspec((_NFH, _L)), cspec((_NFH, _L)),
                  cspec((_L, _NFW)), cspec((_L, _NFW))],
        out_specs=[pl.BlockSpec((1, _NFH, _NFW), lambda g: (g, 0, 0))] * 2,
        out_shape=[jax.ShapeDtypeStruct((n, _NFH, _NFW), jnp.float32)] * 2,
        compiler_params=pltpu.CompilerParams(
            dimension_semantics=("parallel",)),
    )(xt, rr, ri, cr, ci)


def _mix_kernel(xr_ref, xi_ref, hr_ref, hi_ref, h0_ref, yr_ref, yi_ref):
    xr = xr_ref[:]
    xi = xi_ref[:]
    h0 = h0_ref[:]
    accr = jnp.zeros(yr_ref.shape, dtype=jnp.float32)
    acci = jnp.zeros(yr_ref.shape, dtype=jnp.float32)
    for i in range(_D):
        xri = xr[:, i][:, None]
        xii = xi[:, i][:, None]
        hri = (hr_ref[i] + h0[i][:, None, None])[None]
        hii = hi_ref[i][None]
        accr = accr + xri * hri - xii * hii
        acci = acci + xri * hii + xii * hri
    yr_ref[:] = accr
    yi_ref[:] = acci


def _mix(Xr, Xi, Hr, Hi, h0):
    xspec = pl.BlockSpec((4, _D, _TH, _NFW), lambda g: (0, 0, g, 0))
    hspec = pl.BlockSpec((_D, _D, _TH, _NFW), lambda g: (0, 0, g, 0))
    return pl.pallas_call(
        _mix_kernel,
        grid=(_NFH // _TH,),
        in_specs=[xspec, xspec, hspec, hspec,
                  pl.BlockSpec((_D, _D), lambda g: (0, 0))],
        out_specs=[xspec, xspec],
        out_shape=[jax.ShapeDtypeStruct((4, _D, _NFH, _NFW), jnp.float32)] * 2,
        compiler_params=pltpu.CompilerParams(
            dimension_semantics=("parallel",)),
    )(Xr, Xi, Hr, Hi, h0)


def _inv_kernel(yr_ref, yi_ref, vr_ref, vi_ref, wr_ref, wi_ref, y_out):
    yr, yi = yr_ref[0], yi_ref[0]
    zr, zi = _cmm(vr_ref[:], vi_ref[:], yr, yi)
    y_out[0] = (jnp.dot(zr, wr_ref[:], precision=_PREC)
                - jnp.dot(zi, wi_ref[:], precision=_PREC))


def _inv(Yr, Yi, vr, vi, wr, wi):
    n = Yr.shape[0]
    cspec = lambda shape: pl.BlockSpec(shape, lambda g: (0, 0))
    return pl.pallas_call(
        _inv_kernel,
        grid=(n,),
        in_specs=[pl.BlockSpec((1, _NFH, _NFW), lambda g: (g, 0, 0))] * 2
                 + [cspec((_L, _NFH)), cspec((_L, _NFH)),
                    cspec((_NFW, _L)), cspec((_NFW, _L))],
        out_specs=pl.BlockSpec((1, _L, _L), lambda g: (g, 0, 0)),
        out_shape=jax.ShapeDtypeStruct((n, _L, _L), jnp.float32),
        compiler_params=pltpu.CompilerParams(
            dimension_semantics=("parallel",)),
    )(Yr, Yi, vr, vi, wr, wi)


def _constants():
    f_r, f_i = _trig(_L, _L, _L)
    n = jnp.arange(_L, dtype=jnp.int32)
    thw = n.astype(jnp.float32) * jnp.float32(jnp.pi / _L)
    wmr, wmi = jnp.cos(thw)[None, :], -jnp.sin(thw)[None, :]
    e_r = jnp.concatenate([f_r, f_r * wmr - f_i * wmi], axis=0)
    e_i = jnp.concatenate([f_i, f_r * wmi + f_i * wmr], axis=0)
    inv_r, inv_i = f_r / _L, -f_i / _L
    e10_r, e10_i = _trig(_NFH, _L, _NFH)
    m1r, m1i = _cmm(e10_r, e10_i, inv_r, inv_i)
    c1r, c1i = m1r[1::2], m1i[1::2]
    c_r, c_i = _trig(_L, _NFW, _NFH)
    m2r, m2i = _cmm(inv_r, inv_i, c_r, c_i)
    ve_r, ve_i = _trig(_L, _L, _L)
    un = jnp.cos(thw)[:, None], jnp.sin(thw)[:, None]
    vo_r = ve_r * un[0] + ve_i * un[1]
    vo_i = -ve_i * un[0] + ve_r * un[1]
    v_r = jnp.concatenate([ve_r, vo_r], axis=1)
    v_i = jnp.concatenate([-ve_i, vo_i], axis=1)
    w_r, w_i = _trig(_NFW, _L, _NFH)
    w_i = -w_i
    cm = jnp.full((_NFW, 1), 2.0, jnp.float32)
    cm = cm.at[0, 0].set(1.0).at[_NFW - 1, 0].set(1.0) / float(_NFH * _NFH)
    w_r = w_r * cm
    w_i = w_i * cm
    w1_r, w1_i = _trig(_L, 16, _L)
    w2_r, w2_i = _trig(16, _L // 2 + 1, _L)
    return (e_r, e_i, c_r, c_i, c1r, c1i, m2r, m2i, v_r, v_i, w_r, w_i,
            w1_r, w1_i, w2_r, w2_i)


def kernel(x, h0, numerators, denumerator):
    l1, l2 = x.shape[-3], x.shape[-2]
    a = jnp.pad(denumerator, ((0, 0), (0, 0), (1, 0), (1, 0)), constant_values=1.0)
    b0 = jnp.pad(numerators[0], ((0, 0), (0, 0), (1, 0), (1, 0)), constant_values=0.0)
    apad = jnp.pad(a.reshape(_D * _D, 9, 9), ((0, 0), (0, 7), (0, 7)))
    bpad = jnp.pad(b0.reshape(_D * _D, 9, 9), ((0, 0), (0, 7), (0, 7)))

    (er, ei, cr, ci, c1r, c1i, m2r, m2i, vr, vi, wr, wi,
     w1r, w1i, w2r, w2i) = _constants()
    Htr, Hti = _tf(apad, bpad, w1r, w1i, w2r, w2i)
    def _ext(p, sign):
        mirror = jnp.roll(jnp.flip(p, axis=1), 1, axis=1)
        return jnp.concatenate(
            [p, sign * jnp.flip(mirror[:, :, 1:_L // 2], axis=2)], axis=2)
    Htfr = _ext(Htr, 1.0)
    Htfi = _ext(Hti, -1.0)
    Hr, Hi = _hside(Htfr, Htfi, c1r, c1i, m2r, m2i)

    xt = jnp.transpose(x, (0, 3, 1, 2)).reshape(4 * _D, l1, l2)
    Xr, Xi = _xside(xt, er, ei, cr, ci)

    Yr, Yi = _mix(Xr.reshape(4, _D, _NFH, _NFW), Xi.reshape(4, _D, _NFH, _NFW),
                  Hr.reshape(_D, _D, _NFH, _NFW), Hi.reshape(_D, _D, _NFH, _NFW),
                  h0[:, :, 0, 0])

    y = _inv(Yr.reshape(4 * _D, _NFH, _NFW), Yi.reshape(4 * _D, _NFH, _NFW),
             vr, vi, wr, wi)
    return jnp.transpose(y.reshape(4, _D, l1, l2), (0, 2, 3, 1))
